# baseline (speedup 1.0000x reference)
import jax
import jax.numpy as jnp
from jax import lax
from jax.experimental import pallas as pl
from jax.experimental.pallas import tpu as pltpu
from jax.experimental.pallas import tpu_sc as plsc

_DIM = 1024
_SCALE = _DIM ** (-0.5)

_NC = 2
_NS = 16
_NW = _NC * _NS
_LANES = 16

_SC_ROWS = 2048
_SC_TOTAL = _SC_ROWS * _DIM
_PER_W = _SC_TOTAL // _NW
_CHUNK = 16 * _DIM
_NCHUNK = _PER_W // _CHUNK


def _sc_scale_copy(src_hbm, out_hbm, b0, b1, sr0, sr1, sw0, sw1):
    wid = lax.axis_index("s") * _NC + lax.axis_index("c")
    base = wid * _PER_W

    bufs = (b0, b1)
    rsems = (sr0, sr1)
    wsems = (sw0, sw1)

    def chunk_src(k):
        return src_hbm.at[pl.ds(base + k * _CHUNK, _CHUNK)]

    def chunk_dst(k):
        return out_hbm.at[pl.ds(base + k * _CHUNK, _CHUNK)]

    rd = [None] * _NCHUNK
    wr = [None] * _NCHUNK
    rd[0] = pltpu.async_copy(chunk_src(0), bufs[0], rsems[0])
    rd[1] = pltpu.async_copy(chunk_src(1), bufs[1], rsems[1])
    for k in range(_NCHUNK):
        par = k % 2
        rd[k].wait()
        buf = bufs[par]

        @plsc.parallel_loop(0, _CHUNK, _LANES, unroll=8)
        def _scale(i):
            buf[pl.ds(i, _LANES)] = buf[pl.ds(i, _LANES)] * _SCALE

        wr[k] = pltpu.async_copy(buf, chunk_dst(k), wsems[par])
        if k + 2 < _NCHUNK:
            wr[k].wait()
            rd[k + 2] = pltpu.async_copy(chunk_src(k + 2), bufs[par], rsems[par])
    wr[_NCHUNK - 2].wait()
    wr[_NCHUNK - 1].wait()


def _tc_body(e_ref, o_ref):
    o_ref[...] = e_ref[...] * _SCALE


def kernel(x, embed):
    seq_len = x.shape[1]
    tc_rows = seq_len - _SC_ROWS

    mesh = plsc.VectorSubcoreMesh(
        core_axis_name="c", subcore_axis_name="s",
        num_cores=_NC, num_subcores=_NS,
    )
    sc_run = pl.kernel(
        _sc_scale_copy,
        out_type=jax.ShapeDtypeStruct((_SC_TOTAL,), jnp.float32),
        mesh=mesh,
        scratch_types=[
            pltpu.VMEM((_CHUNK,), jnp.float32),
            pltpu.VMEM((_CHUNK,), jnp.float32),
            pltpu.SemaphoreType.DMA,
            pltpu.SemaphoreType.DMA,
            pltpu.SemaphoreType.DMA,
            pltpu.SemaphoreType.DMA,
        ],
    )
    sc_part = sc_run(embed[tc_rows:seq_len].reshape(-1)).reshape(_SC_ROWS, _DIM)

    rows_per_block = 512
    tc_part = pl.pallas_call(
        _tc_body,
        grid=(tc_rows // rows_per_block,),
        in_specs=[pl.BlockSpec((rows_per_block, _DIM), lambda i: (i, 0))],
        out_specs=pl.BlockSpec((rows_per_block, _DIM), lambda i: (i, 0)),
        out_shape=jax.ShapeDtypeStruct((tc_rows, _DIM), jnp.float32),
    )(embed[:tc_rows])

    return jnp.concatenate([tc_part, sc_part], axis=0)


# speedup vs baseline: 2.3673x; 2.3673x over previous
import jax
import jax.numpy as jnp
from jax import lax
from jax.experimental import pallas as pl
from jax.experimental.pallas import tpu as pltpu
from jax.experimental.pallas import tpu_sc as plsc

_DIM = 1024
_SCALE = _DIM ** (-0.5)

_NC = 2
_NS = 16
_NW = _NC * _NS
_LANES = 16

_ROWS = 8192
_ROWS_PER_W = _ROWS // _NW
_CHUNK_ROWS = 32
_NCHUNK = _ROWS_PER_W // _CHUNK_ROWS
_VECS_PER_ROW = _DIM // _LANES


def _sc_scale_copy(src_hbm, out_hbm, b0, b1, sr0, sr1, sw0, sw1):
    wid = lax.axis_index("s") * _NC + lax.axis_index("c")
    base = wid * _ROWS_PER_W

    bufs = (b0, b1)
    rsems = (sr0, sr1)
    wsems = (sw0, sw1)

    def rows(k):
        return pl.ds(base + k * _CHUNK_ROWS, _CHUNK_ROWS)

    rd = [None] * _NCHUNK
    wr = [None] * _NCHUNK
    rd[0] = pltpu.async_copy(src_hbm.at[rows(0)], bufs[0], rsems[0])
    rd[1] = pltpu.async_copy(src_hbm.at[rows(1)], bufs[1], rsems[1])
    for k in range(_NCHUNK):
        par = k % 2
        rd[k].wait()
        buf = bufs[par]

        @plsc.parallel_loop(0, _CHUNK_ROWS * _VECS_PER_ROW, 1, unroll=8)
        def _scale(i):
            r = i // _VECS_PER_ROW
            c = (i % _VECS_PER_ROW) * _LANES
            buf[r, pl.ds(c, _LANES)] = buf[r, pl.ds(c, _LANES)] * _SCALE

        wr[k] = pltpu.async_copy(buf, out_hbm.at[rows(k)], wsems[par])
        if k + 2 < _NCHUNK:
            wr[k].wait()
            rd[k + 2] = pltpu.async_copy(src_hbm.at[rows(k + 2)], bufs[par], rsems[par])
    wr[_NCHUNK - 2].wait()
    wr[_NCHUNK - 1].wait()


def kernel(x, embed):
    seq_len = x.shape[1]
    mesh = plsc.VectorSubcoreMesh(
        core_axis_name="c", subcore_axis_name="s",
        num_cores=_NC, num_subcores=_NS,
    )
    run = pl.kernel(
        _sc_scale_copy,
        out_type=jax.ShapeDtypeStruct((seq_len, _DIM), jnp.float32),
        mesh=mesh,
        scratch_types=[
            pltpu.VMEM((_CHUNK_ROWS, _DIM), jnp.float32),
            pltpu.VMEM((_CHUNK_ROWS, _DIM), jnp.float32),
            pltpu.SemaphoreType.DMA,
            pltpu.SemaphoreType.DMA,
            pltpu.SemaphoreType.DMA,
            pltpu.SemaphoreType.DMA,
        ],
    )
    return run(embed)


# speedup vs baseline: 3.1549x; 1.3327x over previous
import jax
import jax.numpy as jnp
from jax.experimental import pallas as pl

_DIM = 1024
_SCALE = _DIM ** (-0.5)


def _scale_copy_body(e_ref, o_ref):
    o_ref[...] = e_ref[...] * _SCALE


def kernel(x, embed):
    seq_len = x.shape[1]
    rows_per_block = 256
    grid = (seq_len // rows_per_block,)
    return pl.pallas_call(
        _scale_copy_body,
        grid=grid,
        in_specs=[pl.BlockSpec((rows_per_block, _DIM), lambda i: (i, 0))],
        out_specs=pl.BlockSpec((rows_per_block, _DIM), lambda i: (i, 0)),
        out_shape=jax.ShapeDtypeStruct((seq_len, _DIM), jnp.float32),
    )(embed[:seq_len])


# speedup vs baseline: 4.7602x; 1.5089x over previous
import jax
import jax.numpy as jnp
from jax.experimental import pallas as pl

_DIM = 1024
_SCALE = _DIM ** (-0.5)


def _scale_copy_body(e_ref, o_ref):
    o_ref[...] = e_ref[...] * _SCALE


def kernel(x, embed):
    seq_len = x.shape[1]
    rows_per_block = 1024
    grid = (seq_len // rows_per_block,)
    return pl.pallas_call(
        _scale_copy_body,
        grid=grid,
        in_specs=[pl.BlockSpec((rows_per_block, _DIM), lambda i: (i, 0))],
        out_specs=pl.BlockSpec((rows_per_block, _DIM), lambda i: (i, 0)),
        out_shape=jax.ShapeDtypeStruct((seq_len, _DIM), jnp.float32),
    )(embed[:seq_len])


# speedup vs baseline: 5.1361x; 1.0790x over previous
import jax
import jax.numpy as jnp
from jax.experimental import pallas as pl

_DIM = 1024
_SCALE = _DIM ** (-0.5)


def _scale_copy_body(e_ref, o_ref):
    o_ref[...] = e_ref[...] * _SCALE


def kernel(x, embed):
    seq_len = x.shape[1]
    rows_per_block = 2048
    grid = (seq_len // rows_per_block,)
    return pl.pallas_call(
        _scale_copy_body,
        grid=grid,
        in_specs=[pl.BlockSpec((rows_per_block, _DIM), lambda i: (i, 0))],
        out_specs=pl.BlockSpec((rows_per_block, _DIM), lambda i: (i, 0)),
        out_shape=jax.ShapeDtypeStruct((seq_len, _DIM), jnp.float32),
    )(embed[:seq_len])
